# baseline (speedup 1.0000x reference)
import functools

import jax
import jax.numpy as jnp
from jax import lax
from jax.experimental import pallas as pl
from jax.experimental.pallas import tpu as pltpu
from jax.experimental.pallas import tpu_sc as plsc

_NC, _NS = 2, 16
_NW = _NC * _NS
_CH = 128


def _sc_gather(h_V2d, idx_pad, rows_pad, feat):
    chunks_per_w = rows_pad // (_NW * _CH)
    mesh = plsc.VectorSubcoreMesh(core_axis_name="c", subcore_axis_name="s")

    @functools.partial(
        pl.kernel,
        out_type=jax.ShapeDtypeStruct((rows_pad, 128), jnp.float32),
        mesh=mesh,
        scratch_types=[
            pltpu.VMEM((_CH,), jnp.int32),
            pltpu.VMEM((_CH, 128), jnp.float32),
            pltpu.SemaphoreType.DMA,
        ],
    )
    def gather_kernel(table_hbm, idx_hbm, out_hbm, idx_v, rows_v, sem):
        wid = lax.axis_index("s") * _NC + lax.axis_index("c")

        @pl.loop(0, chunks_per_w)
        def _(j):
            base = (wid * chunks_per_w + j) * _CH
            pltpu.sync_copy(idx_hbm.at[pl.ds(base, _CH)], idx_v)
            pltpu.async_copy(table_hbm.at[idx_v], rows_v, sem).wait()
            pltpu.sync_copy(rows_v, out_hbm.at[pl.ds(base, _CH)])

    return gather_kernel(h_V2d, idx_pad)


def _pad_table(h_V2d, feat):
    n = h_V2d.shape[0]
    tile = 2000

    def body(x_ref, o_ref):
        o_ref[...] = jnp.concatenate(
            [x_ref[...], jnp.zeros((tile, 128 - feat), jnp.float32)], axis=1)

    return pl.pallas_call(
        body,
        grid=(n // tile,),
        in_specs=[pl.BlockSpec((tile, feat), lambda i: (i, 0))],
        out_specs=pl.BlockSpec((tile, 128), lambda i: (i, 0)),
        out_shape=jax.ShapeDtypeStruct((n, 128), jnp.float32),
    )(h_V2d)


def _tc_body(gV_ref, gE_ref, hV_ref, W1a_ref, W1b_ref, b1_ref, W2_ref, b2_ref,
             W3_ref, b3_ref, Win_ref, bin_ref, Wout_ref, bout_ref, out_ref,
             *, tile_n, K):
    act = lambda x: 0.5 * x * (1.0 + jax.lax.erf(x * 0.7071067811865476))
    gv = gV_ref[:, :W1a_ref.shape[0]]
    x = gv @ W1a_ref[...] + gE_ref[...] @ W1b_ref[...] + b1_ref[...]
    h = act(x)
    h = act(h @ W2_ref[...] + b2_ref[...])
    m = h @ W3_ref[...] + b3_ref[...]
    dh = jnp.sum(m.reshape(tile_n, K, m.shape[-1]), axis=1) * (1.0 / 30.0)
    hv = hV_ref[...] + dh
    ffn = act(hv @ Win_ref[...] + bin_ref[...]) @ Wout_ref[...] + bout_ref[...]
    out_ref[...] = hv + ffn


def kernel(h_V, h_E, E_idx, W1_w, W1_b, W2_w, W2_b, W3_w, W3_b,
           Win_w, Win_b, Wout_w, Wout_b):
    B, N, H = h_V.shape
    K = h_E.shape[2]
    DE = h_E.shape[3]

    hV2 = h_V.reshape(N, H)
    hE2 = h_E.reshape(N * K, DE)

    rows = N * K
    per_w = _NW * _CH
    rows_pad = ((rows + per_w - 1) // per_w) * per_w
    idx = E_idx.reshape(rows)
    if rows_pad != rows:
        idx = jnp.pad(idx, (0, rows_pad - rows))

    table128 = _pad_table(hV2, H)
    gV = _sc_gather(table128, idx, rows_pad, H)

    W1a = W1_w[:H]
    W1b = W1_w[H:]

    TILE_N = 400
    grid = (N // TILE_N,)

    full = lambda a: pl.BlockSpec(a.shape, lambda i: (0,) * a.ndim)

    out = pl.pallas_call(
        lambda *refs: _tc_body(*refs, tile_n=TILE_N, K=K),
        grid=grid,
        in_specs=[
            pl.BlockSpec((TILE_N * K, 128), lambda i: (i, 0)),
            pl.BlockSpec((TILE_N * K, DE), lambda i: (i, 0)),
            pl.BlockSpec((TILE_N, H), lambda i: (i, 0)),
            full(W1a), full(W1b), full(W1_b.reshape(1, -1)),
            full(W2_w), full(W2_b.reshape(1, -1)),
            full(W3_w), full(W3_b.reshape(1, -1)),
            full(Win_w), full(Win_b.reshape(1, -1)),
            full(Wout_w), full(Wout_b.reshape(1, -1)),
        ],
        out_specs=pl.BlockSpec((TILE_N, H), lambda i: (i, 0)),
        out_shape=jax.ShapeDtypeStruct((N, H), h_V.dtype),
    )(gV, hE2, hV2, W1a, W1b, W1_b.reshape(1, -1), W2_w, W2_b.reshape(1, -1),
      W3_w, W3_b.reshape(1, -1), Win_w, Win_b.reshape(1, -1),
      Wout_w, Wout_b.reshape(1, -1))

    return out.reshape(B, N, H)


# speedup vs baseline: 1.1389x; 1.1389x over previous
import functools

import jax
import jax.numpy as jnp
from jax import lax
from jax.experimental import pallas as pl
from jax.experimental.pallas import tpu as pltpu
from jax.experimental.pallas import tpu_sc as plsc

_NC, _NS = 2, 16
_NW = _NC * _NS
_CH = 128


_SCK = 256
_NB = 2


def _sc_gather(h_V2d, idx_pad, rows_pad, feat):
    chunks_per_w = rows_pad // (_NW * _SCK)
    mesh = plsc.VectorSubcoreMesh(core_axis_name="c", subcore_axis_name="s")

    @functools.partial(
        pl.kernel,
        out_type=jax.ShapeDtypeStruct((rows_pad, 128), jnp.float32),
        mesh=mesh,
        scratch_types=[
            pltpu.VMEM((_NB, _SCK), jnp.int32),
            pltpu.VMEM((_NB, _SCK, 128), jnp.float32),
            pltpu.SemaphoreType.DMA((_NB,)),
            pltpu.SemaphoreType.DMA((_NB,)),
            pltpu.SemaphoreType.DMA((_NB,)),
        ],
    )
    def gather_kernel(table_hbm, idx_hbm, out_hbm, idx_v, rows_v,
                      isem, gsem, ssem):
        wid = lax.axis_index("s") * _NC + lax.axis_index("c")
        base = wid * chunks_per_w

        def idx_copy(c, b):
            return pltpu.make_async_copy(
                idx_hbm.at[pl.ds(c * _SCK, _SCK)], idx_v.at[b], isem.at[b])

        def out_copy(c, b):
            return pltpu.make_async_copy(
                rows_v.at[b], out_hbm.at[pl.ds(c * _SCK, _SCK)], ssem.at[b])

        for b in range(_NB):
            idx_copy(base + b, b).start()

        @pl.loop(0, chunks_per_w, step=_NB)
        def _(t):
            for b in range(_NB):
                cur = base + t + b
                idx_copy(cur, b).wait()

                @pl.when(t + b >= _NB)
                def _():
                    out_copy(cur - _NB, b).wait()

                h0 = pltpu.async_copy(
                    table_hbm.at[idx_v.at[b].at[pl.ds(0, _CH)]],
                    rows_v.at[b].at[pl.ds(0, _CH)], gsem.at[b])
                h1 = pltpu.async_copy(
                    table_hbm.at[idx_v.at[b].at[pl.ds(_CH, _CH)]],
                    rows_v.at[b].at[pl.ds(_CH, _CH)], gsem.at[b])
                h0.wait()
                h1.wait()
                out_copy(cur, b).start()

                @pl.when(t + b + _NB < chunks_per_w)
                def _():
                    idx_copy(cur + _NB, b).start()

        for b in range(_NB):
            out_copy(base + chunks_per_w - _NB + b, b).wait()

    return gather_kernel(h_V2d, idx_pad)


def _pad_table(h_V2d, feat):
    n = h_V2d.shape[0]
    tile = 2000

    def body(x_ref, o_ref):
        o_ref[...] = jnp.concatenate(
            [x_ref[...], jnp.zeros((tile, 128 - feat), jnp.float32)], axis=1)

    return pl.pallas_call(
        body,
        grid=(n // tile,),
        in_specs=[pl.BlockSpec((tile, feat), lambda i: (i, 0))],
        out_specs=pl.BlockSpec((tile, 128), lambda i: (i, 0)),
        out_shape=jax.ShapeDtypeStruct((n, 128), jnp.float32),
    )(h_V2d)


def _tc_body(gV_ref, gE_ref, hV_ref, W1a_ref, W1b_ref, b1_ref, W2_ref, b2_ref,
             W3_ref, b3_ref, Win_ref, bin_ref, Wout_ref, bout_ref, out_ref,
             *, tile_n, K):
    act = lambda x: 0.5 * x * (1.0 + jax.lax.erf(x * 0.7071067811865476))
    gv = gV_ref[:, :W1a_ref.shape[0]]
    x = gv @ W1a_ref[...] + gE_ref[...] @ W1b_ref[...] + b1_ref[...]
    h = act(x)
    h = act(h @ W2_ref[...] + b2_ref[...])
    m = h @ W3_ref[...] + b3_ref[...]
    dh = jnp.sum(m.reshape(tile_n, K, m.shape[-1]), axis=1) * (1.0 / 30.0)
    hv = hV_ref[...] + dh
    ffn = act(hv @ Win_ref[...] + bin_ref[...]) @ Wout_ref[...] + bout_ref[...]
    out_ref[...] = hv + ffn


def kernel(h_V, h_E, E_idx, W1_w, W1_b, W2_w, W2_b, W3_w, W3_b,
           Win_w, Win_b, Wout_w, Wout_b):
    B, N, H = h_V.shape
    K = h_E.shape[2]
    DE = h_E.shape[3]

    hV2 = h_V.reshape(N, H)
    hE2 = h_E.reshape(N * K, DE)

    rows = N * K
    per_super = _NW * _SCK
    n_super = (rows + per_super - 1) // per_super
    n_super += n_super % _NB
    rows_pad = n_super * per_super
    idx = E_idx.reshape(rows)
    if rows_pad != rows:
        idx = jnp.pad(idx, (0, rows_pad - rows))

    table128 = _pad_table(hV2, H)
    gV = _sc_gather(table128, idx, rows_pad, H)

    W1a = W1_w[:H]
    W1b = W1_w[H:]

    TILE_N = 400
    grid = (N // TILE_N,)

    full = lambda a: pl.BlockSpec(a.shape, lambda i: (0,) * a.ndim)

    out = pl.pallas_call(
        lambda *refs: _tc_body(*refs, tile_n=TILE_N, K=K),
        grid=grid,
        in_specs=[
            pl.BlockSpec((TILE_N * K, 128), lambda i: (i, 0)),
            pl.BlockSpec((TILE_N * K, DE), lambda i: (i, 0)),
            pl.BlockSpec((TILE_N, H), lambda i: (i, 0)),
            full(W1a), full(W1b), full(W1_b.reshape(1, -1)),
            full(W2_w), full(W2_b.reshape(1, -1)),
            full(W3_w), full(W3_b.reshape(1, -1)),
            full(Win_w), full(Win_b.reshape(1, -1)),
            full(Wout_w), full(Wout_b.reshape(1, -1)),
        ],
        out_specs=pl.BlockSpec((TILE_N, H), lambda i: (i, 0)),
        out_shape=jax.ShapeDtypeStruct((N, H), h_V.dtype),
    )(gV, hE2, hV2, W1a, W1b, W1_b.reshape(1, -1), W2_w, W2_b.reshape(1, -1),
      W3_w, W3_b.reshape(1, -1), Win_w, Win_b.reshape(1, -1),
      Wout_w, Wout_b.reshape(1, -1))

    return out.reshape(B, N, H)
